# baseline (speedup 1.0000x reference)
import functools

import jax
import jax.numpy as jnp
from jax import lax
from jax.experimental import pallas as pl
from jax.experimental.pallas import tpu as pltpu
from jax.experimental.pallas import tpu_sc as plsc

D = 1024
N_ROWS = 16384
NW = 32
ROWS_PER_W = N_ROWS // NW
C = 16
N_CHUNKS = ROWS_PER_W // C
LANES = 16
CBLKS = D // LANES
NBUF = 4


def _sc_body(x_hbm, fs_hbm, tab_hbm, out_hbm,
             xbuf, sbuf, tbuf, semx, semi, semo):
    wid = lax.axis_index("s") * 2 + lax.axis_index("c")
    base = wid * ROWS_PER_W

    pltpu.sync_copy(tab_hbm, tbuf)

    def in_descs(g, b):
        row0 = base + g * C
        return (
            pltpu.make_async_copy(x_hbm.at[pl.ds(row0, C)], xbuf.at[b],
                                  semx.at[b]),
            pltpu.make_async_copy(fs_hbm.at[pl.ds(row0, C)], sbuf.at[b],
                                  semi.at[b]),
        )

    def out_desc(g, b):
        row0 = base + g * C
        return pltpu.make_async_copy(xbuf.at[b], out_hbm.at[pl.ds(row0, C)],
                                     semo.at[b])

    def start_in(g, b):
        for d in in_descs(g, b):
            d.start()

    def wait_in(g, b):
        for d in in_descs(g, b):
            d.wait()

    def compute(b):
        @plsc.parallel_loop(0, C, 2)
        def row_body(r):
            fa = sbuf[b, r]
            fb = sbuf[b, r + 1]

            @plsc.parallel_loop(0, CBLKS, 1, unroll=8)
            def blk_body(c):
                off = c * LANES
                ta = plsc.load_gather(tbuf, [fa + off])
                tb = plsc.load_gather(tbuf, [fb + off])
                plsc.addupdate(xbuf.at[b, r, pl.ds(off, LANES)], ta)
                plsc.addupdate(xbuf.at[b, r + 1, pl.ds(off, LANES)], tb)

    for g in range(N_CHUNKS):
        in_descs(g, g % NBUF)[0].start()
    for g in range(N_CHUNKS):
        in_descs(g, g % NBUF)[0].wait()


def kernel(x, type_idx, type_embedding):
    B, S, d = x.shape
    x2 = x.reshape(N_ROWS, D)
    idx = type_idx.reshape(N_ROWS).astype(jnp.int32)
    fsplat = idx[:, None] * D + jnp.arange(LANES, dtype=jnp.int32)[None, :]
    tab = type_embedding.reshape(3 * D)

    mesh = plsc.VectorSubcoreMesh(core_axis_name="c", subcore_axis_name="s")
    f = functools.partial(
        pl.kernel,
        out_type=jax.ShapeDtypeStruct((N_ROWS, D), jnp.float32),
        mesh=mesh,
        compiler_params=pltpu.CompilerParams(needs_layout_passes=False),
        scratch_types=[
            pltpu.VMEM((NBUF, C, D), jnp.float32),
            pltpu.VMEM((NBUF, C, LANES), jnp.int32),
            pltpu.VMEM((3 * D,), jnp.float32),
            pltpu.SemaphoreType.DMA((NBUF,)),
            pltpu.SemaphoreType.DMA((NBUF,)),
            pltpu.SemaphoreType.DMA((NBUF,)),
        ],
    )(_sc_body)
    out = f(x2, fsplat, tab)
    return out.reshape(B, S, d)
